# baseline (speedup 1.0000x reference)
import jax
import jax.numpy as jnp
from jax.experimental import pallas as pl
from jax.experimental.pallas import tpu as pltpu

_MARGIN = 0.2
_EPS = 1e-6

_BM = 2048
_BN = 1024


def _finalize(ap, an, rt, w, nacc_ref):
    msp = jnp.max(ap, axis=1, keepdims=True).astype(jnp.float32) + rt
    msn = jnp.min(an, axis=1, keepdims=True).astype(jnp.float32) + rt
    dp = jnp.sqrt(jnp.maximum(msp, 0.0))
    dn = jnp.sqrt(jnp.maximum(msn, 0.0))
    nacc_ref[...] += jnp.maximum(dp - dn + _MARGIN, 0.0) * w


def _body(a_ref, bt_ref, tcol_ref, trow_prev_ref, trow_ref, num_ref, cnt_ref,
          cp_ref, cn_ref, ap_ref, an_ref, rt_ref, nacc_ref, cnt1_ref):
    i = pl.program_id(0)
    nsteps = pl.num_programs(0)
    d_dim = a_ref.shape[1]
    b_dim = bt_ref.shape[1]
    ninf = jnp.asarray(-jnp.inf, jnp.bfloat16)

    @pl.when(i == 0)
    def _init():
        btf = bt_ref[...].astype(jnp.float32)
        colterm = jnp.sum(btf * (btf - (2.0 * _EPS)), axis=0, keepdims=True)
        colterm = colterm + (d_dim * _EPS * _EPS)
        posm = tcol_ref[...] == 1
        cp_ref[...] = jnp.where(posm, colterm, -jnp.inf).astype(jnp.bfloat16)
        cn_ref[...] = jnp.where(posm, jnp.inf, colterm).astype(jnp.bfloat16)
        nacc_ref[...] = jnp.zeros_like(nacc_ref)
        cnt1_ref[...] = jnp.sum(posm.astype(jnp.float32), keepdims=True)
        ap_ref[...] = jnp.full_like(ap_ref, ninf)
        an_ref[...] = jnp.full_like(an_ref, -ninf)
        rt_ref[...] = jnp.zeros_like(rt_ref)

    w_prev = (trow_prev_ref[...] == 1).astype(jnp.float32)
    _finalize(ap_ref[...], an_ref[...], rt_ref[...], w_prev, nacc_ref)

    a = a_ref[...]
    am2 = a * jnp.bfloat16(-2.0)
    af = a.astype(jnp.float32)
    rowterm = jnp.sum(af * (af + (2.0 * _EPS)), axis=1, keepdims=True)

    acc_p = jnp.full((_BM, 128), ninf, jnp.bfloat16)
    acc_n = jnp.full((_BM, 128), -ninf, jnp.bfloat16)
    for c in range(b_dim // _BN):
        btc = bt_ref[:, c * _BN:(c + 1) * _BN]
        t2 = jax.lax.dot_general(am2, btc, (((1,), (0,)), ((), ())),
                                 preferred_element_type=jnp.float32
                                 ).astype(jnp.bfloat16)
        cp = cp_ref[0:1, c * _BN:(c + 1) * _BN]
        cn = cn_ref[0:1, c * _BN:(c + 1) * _BN]
        tp = t2 + cp
        tn = t2 + cn
        for s in range(_BN // 128):
            acc_p = jnp.maximum(acc_p, tp[:, s * 128:(s + 1) * 128])
            acc_n = jnp.minimum(acc_n, tn[:, s * 128:(s + 1) * 128])

    ap_ref[...] = acc_p
    an_ref[...] = acc_n
    rt_ref[...] = rowterm

    @pl.when(i == nsteps - 1)
    def _fin():
        w = (trow_ref[...] == 1).astype(jnp.float32)
        _finalize(acc_p, acc_n, rowterm, w, nacc_ref)
        num_ref[...] = jnp.sum(nacc_ref[...], keepdims=True)
        cnt_ref[...] = cnt1_ref[...]


def kernel(emb1, emb2, target):
    b_dim, d_dim = emb1.shape
    nb = b_dim // _BM
    tgt = target.astype(jnp.int32)
    a_bf = emb1.astype(jnp.bfloat16)
    bt_bf = emb2.T.astype(jnp.bfloat16)
    tcol = tgt.reshape(1, b_dim)
    trow = tgt.reshape(b_dim, 1)

    num, cnt = pl.pallas_call(
        _body,
        grid=(nb,),
        in_specs=[
            pl.BlockSpec((_BM, d_dim), lambda i: (i, 0)),
            pl.BlockSpec((d_dim, b_dim), lambda i: (0, 0)),
            pl.BlockSpec((1, b_dim), lambda i: (0, 0)),
            pl.BlockSpec((_BM, 1), lambda i: (jnp.maximum(i - 1, 0), 0)),
            pl.BlockSpec((_BM, 1), lambda i: (i, 0)),
        ],
        out_specs=[
            pl.BlockSpec((1, 1), lambda i: (0, 0)),
            pl.BlockSpec((1, 1), lambda i: (0, 0)),
        ],
        out_shape=[
            jax.ShapeDtypeStruct((1, 1), jnp.float32),
            jax.ShapeDtypeStruct((1, 1), jnp.float32),
        ],
        scratch_shapes=[
            pltpu.VMEM((1, b_dim), jnp.bfloat16),
            pltpu.VMEM((1, b_dim), jnp.bfloat16),
            pltpu.VMEM((_BM, 128), jnp.bfloat16),
            pltpu.VMEM((_BM, 128), jnp.bfloat16),
            pltpu.VMEM((_BM, 1), jnp.float32),
            pltpu.VMEM((_BM, 1), jnp.float32),
            pltpu.VMEM((1, 1), jnp.float32),
        ],
        compiler_params=pltpu.CompilerParams(
            dimension_semantics=("arbitrary",),
            allow_input_fusion=(True, True, True, True, True),
            vmem_limit_bytes=48 * 1024 * 1024,
        ),
    )(a_bf, bt_bf, tcol, trow, trow)

    return num[0, 0] / cnt[0, 0]


# speedup vs baseline: 1.0242x; 1.0242x over previous
import jax
import jax.numpy as jnp
from jax.experimental import pallas as pl
from jax.experimental.pallas import tpu as pltpu

_MARGIN = 0.2
_EPS = 1e-6

_BM = 2048
_BN = 1024


def _body(a_ref, bt_ref, tcol_ref, trow_ref, num_ref, cnt_ref,
          cp_ref, cn_ref, nacc_ref, cacc_ref):
    i = pl.program_id(0)
    nsteps = pl.num_programs(0)
    d_dim = a_ref.shape[1]
    b_dim = bt_ref.shape[1]

    @pl.when(i == 0)
    def _init():
        btf = bt_ref[...].astype(jnp.float32)
        colterm = jnp.sum(btf * (btf - (2.0 * _EPS)), axis=0, keepdims=True)
        colterm = colterm + (d_dim * _EPS * _EPS)
        posm = tcol_ref[...] == 1
        cp_ref[...] = jnp.where(posm, colterm, -jnp.inf).astype(jnp.bfloat16)
        cn_ref[...] = jnp.where(posm, jnp.inf, colterm).astype(jnp.bfloat16)
        nacc_ref[...] = jnp.zeros_like(nacc_ref)
        cacc_ref[...] = jnp.zeros_like(cacc_ref)

    a = a_ref[...]
    am2 = a * jnp.bfloat16(-2.0)
    af = a.astype(jnp.float32)
    rowterm = jnp.sum(af * (af + (2.0 * _EPS)), axis=1, keepdims=True)

    ninf = jnp.asarray(-jnp.inf, jnp.bfloat16)
    acc_p = jnp.full((_BM, 128), ninf, jnp.bfloat16)
    acc_n = jnp.full((_BM, 128), -ninf, jnp.bfloat16)
    for c in range(b_dim // _BN):
        btc = bt_ref[:, c * _BN:(c + 1) * _BN]
        t2 = jax.lax.dot_general(am2, btc, (((1,), (0,)), ((), ())),
                                 preferred_element_type=jnp.float32
                                 ).astype(jnp.bfloat16)
        cp = cp_ref[0:1, c * _BN:(c + 1) * _BN]
        cn = cn_ref[0:1, c * _BN:(c + 1) * _BN]
        tp = t2 + cp
        tn = t2 + cn
        for s in range(_BN // 128):
            acc_p = jnp.maximum(acc_p, tp[:, s * 128:(s + 1) * 128])
            acc_n = jnp.minimum(acc_n, tn[:, s * 128:(s + 1) * 128])

    msp = jnp.max(acc_p, axis=1, keepdims=True).astype(jnp.float32) + rowterm
    msn = jnp.min(acc_n, axis=1, keepdims=True).astype(jnp.float32) + rowterm
    dp = jnp.sqrt(jnp.maximum(msp, 0.0))
    dn = jnp.sqrt(jnp.maximum(msn, 0.0))
    w = (trow_ref[...] == 1).astype(jnp.float32)
    nacc_ref[...] += jnp.maximum(dp - dn + _MARGIN, 0.0) * w
    cacc_ref[...] += w

    @pl.when(i == nsteps - 1)
    def _fin():
        num_ref[...] = jnp.sum(nacc_ref[...], keepdims=True)
        cnt_ref[...] = jnp.sum(cacc_ref[...], keepdims=True)


def kernel(emb1, emb2, target):
    b_dim, d_dim = emb1.shape
    nb = b_dim // _BM
    tgt = target.astype(jnp.int32)
    a_bf = emb1.astype(jnp.bfloat16)
    bt_bf = emb2.T.astype(jnp.bfloat16)
    tcol = tgt.reshape(1, b_dim)
    trow = tgt.reshape(b_dim, 1)

    num, cnt = pl.pallas_call(
        _body,
        grid=(nb,),
        in_specs=[
            pl.BlockSpec((_BM, d_dim), lambda i: (i, 0)),
            pl.BlockSpec((d_dim, b_dim), lambda i: (0, 0)),
            pl.BlockSpec((1, b_dim), lambda i: (0, 0)),
            pl.BlockSpec((_BM, 1), lambda i: (i, 0)),
        ],
        out_specs=[
            pl.BlockSpec((1, 1), lambda i: (0, 0)),
            pl.BlockSpec((1, 1), lambda i: (0, 0)),
        ],
        out_shape=[
            jax.ShapeDtypeStruct((1, 1), jnp.float32),
            jax.ShapeDtypeStruct((1, 1), jnp.float32),
        ],
        scratch_shapes=[
            pltpu.VMEM((1, b_dim), jnp.bfloat16),
            pltpu.VMEM((1, b_dim), jnp.bfloat16),
            pltpu.VMEM((_BM, 1), jnp.float32),
            pltpu.VMEM((_BM, 1), jnp.float32),
        ],
        compiler_params=pltpu.CompilerParams(
            dimension_semantics=("arbitrary",),
            allow_input_fusion=(True, True, True, True),
            vmem_limit_bytes=48 * 1024 * 1024,
        ),
    )(a_bf, bt_bf, tcol, trow)

    return num[0, 0] / cnt[0, 0]


# speedup vs baseline: 1.0354x; 1.0110x over previous
import jax
import jax.numpy as jnp
from jax.experimental import pallas as pl
from jax.experimental.pallas import tpu as pltpu

_MARGIN = 0.2
_EPS = 1e-6

_BM = 4096
_BN = 1024


def _body(a_ref, bt_ref, tcol_ref, trow_ref, num_ref, cnt_ref,
          cp_ref, cn_ref, nacc_ref, cacc_ref):
    i = pl.program_id(0)
    nsteps = pl.num_programs(0)
    d_dim = a_ref.shape[1]
    b_dim = bt_ref.shape[1]

    @pl.when(i == 0)
    def _init():
        btf = bt_ref[...].astype(jnp.float32)
        colterm = jnp.sum(btf * (btf - (2.0 * _EPS)), axis=0, keepdims=True)
        colterm = colterm + (d_dim * _EPS * _EPS)
        posm = tcol_ref[...] == 1
        cp_ref[...] = jnp.where(posm, colterm, -jnp.inf).astype(jnp.bfloat16)
        cn_ref[...] = jnp.where(posm, jnp.inf, colterm).astype(jnp.bfloat16)
        nacc_ref[...] = jnp.zeros_like(nacc_ref)
        cacc_ref[...] = jnp.zeros_like(cacc_ref)

    a = a_ref[...]
    am2 = a * jnp.bfloat16(-2.0)
    af = a.astype(jnp.float32)
    rowterm = jnp.sum(af * (af + (2.0 * _EPS)), axis=1, keepdims=True)

    ninf = jnp.asarray(-jnp.inf, jnp.bfloat16)
    acc_p = jnp.full((_BM, 128), ninf, jnp.bfloat16)
    acc_n = jnp.full((_BM, 128), -ninf, jnp.bfloat16)
    for c in range(b_dim // _BN):
        btc = bt_ref[:, c * _BN:(c + 1) * _BN]
        t2 = jax.lax.dot_general(am2, btc, (((1,), (0,)), ((), ())),
                                 preferred_element_type=jnp.float32
                                 ).astype(jnp.bfloat16)
        cp = cp_ref[0:1, c * _BN:(c + 1) * _BN]
        cn = cn_ref[0:1, c * _BN:(c + 1) * _BN]
        tp = t2 + cp
        tn = t2 + cn
        for s in range(_BN // 128):
            acc_p = jnp.maximum(acc_p, tp[:, s * 128:(s + 1) * 128])
            acc_n = jnp.minimum(acc_n, tn[:, s * 128:(s + 1) * 128])

    msp = jnp.max(acc_p, axis=1, keepdims=True).astype(jnp.float32) + rowterm
    msn = jnp.min(acc_n, axis=1, keepdims=True).astype(jnp.float32) + rowterm
    dp = jnp.sqrt(jnp.maximum(msp, 0.0))
    dn = jnp.sqrt(jnp.maximum(msn, 0.0))
    w = (trow_ref[...] == 1).astype(jnp.float32)
    nacc_ref[...] += jnp.maximum(dp - dn + _MARGIN, 0.0) * w
    cacc_ref[...] += w

    @pl.when(i == nsteps - 1)
    def _fin():
        num_ref[...] = jnp.sum(nacc_ref[...], keepdims=True)
        cnt_ref[...] = jnp.sum(cacc_ref[...], keepdims=True)


def kernel(emb1, emb2, target):
    b_dim, d_dim = emb1.shape
    nb = b_dim // _BM
    tgt = target.astype(jnp.int32)
    a_bf = emb1.astype(jnp.bfloat16)
    bt_bf = emb2.T.astype(jnp.bfloat16)
    tcol = tgt.reshape(1, b_dim)
    trow = tgt.reshape(b_dim, 1)

    num, cnt = pl.pallas_call(
        _body,
        grid=(nb,),
        in_specs=[
            pl.BlockSpec((_BM, d_dim), lambda i: (i, 0)),
            pl.BlockSpec((d_dim, b_dim), lambda i: (0, 0)),
            pl.BlockSpec((1, b_dim), lambda i: (0, 0)),
            pl.BlockSpec((_BM, 1), lambda i: (i, 0)),
        ],
        out_specs=[
            pl.BlockSpec((1, 1), lambda i: (0, 0)),
            pl.BlockSpec((1, 1), lambda i: (0, 0)),
        ],
        out_shape=[
            jax.ShapeDtypeStruct((1, 1), jnp.float32),
            jax.ShapeDtypeStruct((1, 1), jnp.float32),
        ],
        scratch_shapes=[
            pltpu.VMEM((1, b_dim), jnp.bfloat16),
            pltpu.VMEM((1, b_dim), jnp.bfloat16),
            pltpu.VMEM((_BM, 1), jnp.float32),
            pltpu.VMEM((_BM, 1), jnp.float32),
        ],
        compiler_params=pltpu.CompilerParams(
            dimension_semantics=("arbitrary",),
            allow_input_fusion=(True, True, True, True),
            vmem_limit_bytes=48 * 1024 * 1024,
        ),
    )(a_bf, bt_bf, tcol, trow)

    return num[0, 0] / cnt[0, 0]
